# baseline (speedup 1.0000x reference)
import functools

import jax
import jax.numpy as jnp
from jax import lax
from jax.experimental import pallas as pl
from jax.experimental.pallas import tpu as pltpu
from jax.experimental.pallas import tpu_sc as plsc

BATCH = 16384
EMBED_DIM = 64
LANES = 16
NUM_WORKERS = 32
B_PER_W = BATCH // NUM_WORKERS
IDX_CHUNK = 128
N_CHUNKS = B_PER_W // IDX_CHUNK
GROUPS = B_PER_W // LANES


def _sc_body(users_hbm, items_hbm, uemb_hbm, iemb_hbm, bias_hbm, out_hbm,
             uidx_v, iidx_v, urows_v, irows_v, bias_v, out_v, sem):
    wid = lax.axis_index("s") * 2 + lax.axis_index("c")
    base = wid * B_PER_W

    pltpu.sync_copy(users_hbm.at[pl.ds(base, B_PER_W)], uidx_v)
    pltpu.sync_copy(items_hbm.at[pl.ds(base, B_PER_W)], iidx_v)

    copies = []
    for j in range(N_CHUNKS):
        sl = pl.ds(j * IDX_CHUNK, IDX_CHUNK)
        copies.append(pltpu.async_copy(
            uemb_hbm.at[uidx_v.at[sl]], urows_v.at[sl, :], sem))
        copies.append(pltpu.async_copy(
            iemb_hbm.at[iidx_v.at[sl]], irows_v.at[sl, :], sem))
        copies.append(pltpu.async_copy(
            bias_hbm.at[iidx_v.at[sl]], bias_v.at[sl], sem))
    for c in copies:
        c.wait()

    lanes = lax.iota(jnp.int32, LANES)

    def g_body(g, carry):
        rows = lanes + g * LANES
        acc = bias_v[pl.ds(g * LANES, LANES)]
        for d in range(EMBED_DIM):
            dd = jnp.full((LANES,), d, jnp.int32)
            u = plsc.load_gather(urows_v, [rows, dd])
            iv = plsc.load_gather(irows_v, [rows, dd])
            acc = acc + u * iv
        out_v[pl.ds(g * LANES, LANES)] = acc
        return carry

    lax.fori_loop(0, GROUPS, g_body, 0)
    pltpu.sync_copy(out_v, out_hbm.at[pl.ds(base, B_PER_W)])


@jax.jit
def kernel(users, items, user_embeddings, item_embeddings, item_biases):
    mesh = plsc.VectorSubcoreMesh(core_axis_name="c", subcore_axis_name="s")
    f = pl.kernel(
        _sc_body,
        out_type=jax.ShapeDtypeStruct((BATCH,), jnp.float32),
        mesh=mesh,
        compiler_params=pltpu.CompilerParams(
            needs_layout_passes=False, use_tc_tiling_on_sc=False),
        scratch_types=[
            pltpu.VMEM((B_PER_W,), jnp.int32),
            pltpu.VMEM((B_PER_W,), jnp.int32),
            pltpu.VMEM((B_PER_W, EMBED_DIM), jnp.float32),
            pltpu.VMEM((B_PER_W, EMBED_DIM), jnp.float32),
            pltpu.VMEM((B_PER_W,), jnp.float32),
            pltpu.VMEM((B_PER_W,), jnp.float32),
            pltpu.SemaphoreType.DMA,
        ],
    )
    return f(users.astype(jnp.int32), items.astype(jnp.int32),
             user_embeddings, item_embeddings, item_biases.reshape(-1))
